# baseline (speedup 1.0000x reference)
import functools

import jax
import jax.numpy as jnp
from jax import lax
from jax.experimental import pallas as pl
from jax.experimental.pallas import tpu as pltpu
from jax.experimental.pallas import tpu_sc as plsc

EMBED = 64
BATCH = 16384
PAIR_ROWS = 500000

NC = 2
NS = 16
NW = NC * NS
B_PER_W = BATCH // NW
VL = 16

_mesh = plsc.VectorSubcoreMesh(core_axis_name="c", subcore_axis_name="s")


@functools.partial(
    pl.kernel,
    mesh=_mesh,
    compiler_params=pltpu.CompilerParams(needs_layout_passes=False),
    out_type=[
        jax.ShapeDtypeStruct((EMBED, BATCH), jnp.float32),
        jax.ShapeDtypeStruct((EMBED, BATCH), jnp.float32),
        jax.ShapeDtypeStruct((EMBED, BATCH), jnp.float32),
    ],
    scratch_types=[
        pltpu.VMEM((B_PER_W,), jnp.int32),
        pltpu.VMEM((B_PER_W, 2 * EMBED), jnp.float32),
        pltpu.VMEM((EMBED, B_PER_W), jnp.float32),
        pltpu.SemaphoreType.DMA,
        pltpu.SemaphoreType.DMA,
    ],
)
def _gather3(users_hbm, pos_hbm, neg_hbm, uemb_hbm, iemb_hbm,
             out_u, out_p, out_n,
             idx_v, pairs, out_st, sem_g, sem_s):
    wid = lax.axis_index("s") * NC + lax.axis_index("c")
    base = wid * B_PER_W
    out_sl = pl.ds(base, B_PER_W)

    def one_table(idx_hbm, tab_hbm, out_hbm):
        pltpu.sync_copy(idx_hbm.at[pl.ds(base, B_PER_W)], idx_v)

        def fire(m, _):
            ev = idx_v[pl.ds(m * VL, VL)]
            pltpu.async_copy(tab_hbm.at[lax.shift_right_logical(ev, 1)],
                             pairs.at[pl.ds(m * VL, VL), :], sem_g)

        lax.fori_loop(0, B_PER_W // VL, fire, None, unroll=8)

        def drain(m, _):
            pltpu.make_async_copy(
                tab_hbm.at[pl.ds(0, VL)], pairs.at[pl.ds(m * VL, VL), :],
                sem_g).wait()

        lax.fori_loop(0, B_PER_W // VL, drain, None, unroll=8)

        lanes = lax.iota(jnp.int32, VL)

        def extract(m, _):
            i0 = m * VL
            ev = idx_v[pl.ds(i0, VL)]
            cols0 = lax.mul(lax.bitwise_and(ev, 1), EMBED)
            rows = lax.add(lax.broadcast(i0, (VL,)), lanes)
            for c in range(EMBED):
                v = plsc.load_gather(pairs, [rows, lax.add(cols0, c)])
                out_st[c, pl.ds(i0, VL)] = v

        lax.fori_loop(0, B_PER_W // VL, extract, None)

        pltpu.async_copy(out_st, out_hbm.at[:, out_sl], sem_s).wait()

    one_table(users_hbm, uemb_hbm, out_u)
    one_table(pos_hbm, iemb_hbm, out_p)
    one_table(neg_hbm, iemb_hbm, out_n)


def kernel(users, pos_items, neg_items, _, user_emb, item_emb):
    u = users.astype(jnp.int32)
    p = pos_items.astype(jnp.int32)
    n = neg_items.astype(jnp.int32)
    ut = user_emb.reshape(PAIR_ROWS, 2 * EMBED)
    it = item_emb.reshape(PAIR_ROWS, 2 * EMBED)
    out_u, out_p, out_n = _gather3(u, p, n, ut, it)
    return out_u.T, out_p.T, out_n.T, _


# speedup vs baseline: 1.0265x; 1.0265x over previous
import functools

import jax
import jax.numpy as jnp
from jax import lax
from jax.experimental import pallas as pl
from jax.experimental.pallas import tpu as pltpu
from jax.experimental.pallas import tpu_sc as plsc

EMBED = 64
BATCH = 16384
PAIR_ROWS = 500000

NC = 2
NS = 16
NW = NC * NS
B_PER_W = BATCH // NW
VL = 16

_mesh = plsc.VectorSubcoreMesh(core_axis_name="c", subcore_axis_name="s")


def _tile_body(idx_hbm, tab_hbm, out_hbm, idx_v, pairs, out_st, sem_g, sem_s,
               base, out_sl):
    pltpu.sync_copy(idx_hbm.at[pl.ds(base, B_PER_W)], idx_v)

    def fire(m, _):
        ev = idx_v[pl.ds(m * VL, VL)]
        pltpu.async_copy(tab_hbm.at[lax.shift_right_logical(ev, 1)],
                         pairs.at[pl.ds(m * VL, VL), :], sem_g)

    lax.fori_loop(0, B_PER_W // VL, fire, None, unroll=8)

    def drain(m, _):
        pltpu.make_async_copy(
            tab_hbm.at[pl.ds(0, VL)], pairs.at[pl.ds(m * VL, VL), :],
            sem_g).wait()

    lax.fori_loop(0, B_PER_W // VL, drain, None, unroll=8)

    lanes = lax.iota(jnp.int32, VL)

    def extract(m, _):
        i0 = m * VL
        ev = idx_v[pl.ds(i0, VL)]
        cols0 = lax.mul(lax.bitwise_and(ev, 1), EMBED)
        rows = lax.add(lax.broadcast(i0, (VL,)), lanes)
        for c in range(EMBED):
            v = plsc.load_gather(pairs, [rows, lax.add(cols0, c)])
            out_st[c, pl.ds(i0, VL)] = v

    lax.fori_loop(0, B_PER_W // VL, extract, None)

    pltpu.async_copy(out_st, out_hbm.at[:, out_sl], sem_s).wait()


@functools.partial(
    pl.kernel,
    mesh=_mesh,
    compiler_params=pltpu.CompilerParams(needs_layout_passes=False),
    out_type=jax.ShapeDtypeStruct((EMBED, BATCH), jnp.float32),
    scratch_types=[
        pltpu.VMEM((B_PER_W,), jnp.int32),
        pltpu.VMEM((B_PER_W, 2 * EMBED), jnp.float32),
        pltpu.VMEM((EMBED, B_PER_W), jnp.float32),
        pltpu.SemaphoreType.DMA,
        pltpu.SemaphoreType.DMA,
    ],
)
def _gather_user(users_hbm, uemb_hbm, out_u, idx_v, pairs, out_st,
                 sem_g, sem_s):
    wid = lax.axis_index("s") * NC + lax.axis_index("c")
    base = wid * B_PER_W
    _tile_body(users_hbm, uemb_hbm, out_u, idx_v, pairs, out_st, sem_g, sem_s,
               base, pl.ds(base, B_PER_W))


@functools.partial(
    pl.kernel,
    mesh=_mesh,
    compiler_params=pltpu.CompilerParams(needs_layout_passes=False),
    out_type=[
        jax.ShapeDtypeStruct((EMBED, BATCH), jnp.float32),
        jax.ShapeDtypeStruct((EMBED, BATCH), jnp.float32),
    ],
    scratch_types=[
        pltpu.VMEM((B_PER_W,), jnp.int32),
        pltpu.VMEM((B_PER_W, 2 * EMBED), jnp.float32),
        pltpu.VMEM((EMBED, B_PER_W), jnp.float32),
        pltpu.SemaphoreType.DMA,
        pltpu.SemaphoreType.DMA,
    ],
)
def _gather_items(pos_hbm, neg_hbm, iemb_hbm, out_p, out_n,
                  idx_v, pairs, out_st, sem_g, sem_s):
    wid = lax.axis_index("s") * NC + lax.axis_index("c")
    base = wid * B_PER_W
    out_sl = pl.ds(base, B_PER_W)
    _tile_body(pos_hbm, iemb_hbm, out_p, idx_v, pairs, out_st, sem_g, sem_s,
               base, out_sl)
    _tile_body(neg_hbm, iemb_hbm, out_n, idx_v, pairs, out_st, sem_g, sem_s,
               base, out_sl)


def kernel(users, pos_items, neg_items, _, user_emb, item_emb):
    u = users.astype(jnp.int32)
    p = pos_items.astype(jnp.int32)
    n = neg_items.astype(jnp.int32)
    ut = user_emb.reshape(PAIR_ROWS, 2 * EMBED)
    it = item_emb.reshape(PAIR_ROWS, 2 * EMBED)
    out_u = _gather_user(u, ut)
    out_p, out_n = _gather_items(p, n, it)
    return out_u.T, out_p.T, out_n.T, _
